# Initial kernel scaffold; baseline (speedup 1.0000x reference)
import functools

import jax
import jax.numpy as jnp
from jax.experimental import pallas as pl
from jax.experimental.pallas import tpu as pltpu

N = 10000
D = 128
M_BLK = 1000
K_BLK = 2000
M_BLOCKS = N // M_BLK
K_BLOCKS = N // K_BLK


def _row_norm(v):
    return jnp.sqrt(jnp.sum(v * v, axis=-1, keepdims=True))


def _hypagg_kernel(x_ref, adj_ref, out_ref, xt_ref, acc_ref):
    m = pl.program_id(0)
    k = pl.program_id(1)

    @pl.when(m == 0)
    def _compute_tangent():
        xs = x_ref[pl.ds(k * K_BLK, K_BLK), :]
        norm = jnp.maximum(_row_norm(xs), 1e-15)
        cn = jnp.clip(norm, -1.0 + 1e-7, 1.0 - 1e-7)
        artanh = 0.5 * (jnp.log1p(cn) - jnp.log1p(-cn))
        xt_ref[pl.ds(k * K_BLK, K_BLK), :] = xs * (artanh / norm)

    @pl.when(k == 0)
    def _zero_acc():
        acc_ref[...] = jnp.zeros_like(acc_ref)

    acc_ref[...] += jnp.dot(
        adj_ref[...],
        xt_ref[pl.ds(k * K_BLK, K_BLK), :],
        preferred_element_type=jnp.float32,
    )

    @pl.when(k == K_BLOCKS - 1)
    def _epilogue():
        s = acc_ref[...]
        norm = jnp.maximum(_row_norm(s), 1e-15)
        e = s * (jnp.tanh(norm) / norm)
        maxnorm = 1.0 - 4e-3
        enorm = jnp.maximum(_row_norm(e), 1e-15)
        out_ref[...] = jnp.where(enorm > maxnorm, e * (maxnorm / enorm), e)


@functools.partial(jax.jit, static_argnames=())
def kernel(x, adj):
    out = pl.pallas_call(
        _hypagg_kernel,
        grid=(M_BLOCKS, K_BLOCKS),
        in_specs=[
            pl.BlockSpec((N, D), lambda m, k: (0, 0)),
            pl.BlockSpec((M_BLK, K_BLK), lambda m, k: (m, k)),
        ],
        out_specs=pl.BlockSpec((M_BLK, D), lambda m, k: (m, 0)),
        out_shape=jax.ShapeDtypeStruct((N, D), jnp.float32),
        scratch_shapes=[
            pltpu.VMEM((N, D), jnp.float32),
            pltpu.VMEM((M_BLK, D), jnp.float32),
        ],
        compiler_params=pltpu.CompilerParams(
            dimension_semantics=("arbitrary", "arbitrary"),
        ),
    )(x, adj)
    return out[None, ...]


# speedup vs baseline: 1.1554x; 1.1554x over previous
import functools

import jax
import jax.numpy as jnp
from jax.experimental import pallas as pl
from jax.experimental.pallas import tpu as pltpu

N = 10000
D = 128
M_BLK = 400
M_BLOCKS = N // M_BLK


def _row_norm(v):
    return jnp.sqrt(jnp.sum(v * v, axis=-1, keepdims=True))


def _hypagg_kernel(x_ref, adj_ref, out_ref, xt_ref):
    m = pl.program_id(0)

    @pl.when(m == 0)
    def _compute_tangent():
        xs = x_ref[...]
        norm = jnp.maximum(_row_norm(xs), 1e-15)
        cn = jnp.clip(norm, -1.0 + 1e-7, 1.0 - 1e-7)
        artanh = 0.5 * (jnp.log1p(cn) - jnp.log1p(-cn))
        xt_ref[...] = xs * (artanh / norm)

    s = jnp.dot(adj_ref[...], xt_ref[...], preferred_element_type=jnp.float32)
    norm = jnp.maximum(_row_norm(s), 1e-15)
    e = s * (jnp.tanh(norm) / norm)
    maxnorm = 1.0 - 4e-3
    enorm = jnp.maximum(_row_norm(e), 1e-15)
    out_ref[...] = jnp.where(enorm > maxnorm, e * (maxnorm / enorm), e)


@functools.partial(jax.jit, static_argnames=())
def kernel(x, adj):
    out = pl.pallas_call(
        _hypagg_kernel,
        grid=(M_BLOCKS,),
        in_specs=[
            pl.BlockSpec((N, D), lambda m: (0, 0)),
            pl.BlockSpec((M_BLK, N), lambda m: (m, 0)),
        ],
        out_specs=pl.BlockSpec((M_BLK, D), lambda m: (m, 0)),
        out_shape=jax.ShapeDtypeStruct((N, D), jnp.float32),
        scratch_shapes=[
            pltpu.VMEM((N, D), jnp.float32),
        ],
        compiler_params=pltpu.CompilerParams(
            dimension_semantics=("arbitrary",),
        ),
    )(x, adj)
    return out[None, ...]
